# baseline (speedup 1.0000x reference)
import jax
import jax.numpy as jnp
from jax import lax
from jax.experimental import pallas as pl
from jax.experimental.pallas import tpu as pltpu

_CH = 1000
_NBUF = 6

_DN = (((0,), (1,)), ((), ()))


def _gates_body(zl_hbm, zh_hbm, w1t_ref, b1_ref, w2t_ref, b2_ref,
                ol_ref, oh_ref, bufl, bufh, sems):
    n = zl_hbm.shape[0]
    nch = n // _CH

    def start(i, slot):
        pltpu.make_async_copy(
            zl_hbm.at[pl.ds(i * _CH, _CH), :], bufl.at[slot], sems.at[0, slot]
        ).start()
        pltpu.make_async_copy(
            zh_hbm.at[pl.ds(i * _CH, _CH), :], bufh.at[slot], sems.at[1, slot]
        ).start()

    for s in range(_NBUF - 1):
        start(s, s)

    def body(i, carry):
        slot = jax.lax.rem(i, _NBUF)
        nxt = i + (_NBUF - 1)

        @pl.when(nxt < nch)
        def _():
            start(nxt, jax.lax.rem(nxt, _NBUF))

        pltpu.make_async_copy(
            zl_hbm.at[pl.ds(i * _CH, _CH), :], bufl.at[slot], sems.at[0, slot]
        ).wait()
        pltpu.make_async_copy(
            zh_hbm.at[pl.ds(i * _CH, _CH), :], bufh.at[slot], sems.at[1, slot]
        ).wait()

        ol_ref[i] = (
            lax.dot_general(w1t_ref[...], bufl[slot], _DN,
                            preferred_element_type=jnp.float32)
            + b1_ref[...]
        )
        oh_ref[i] = (
            lax.dot_general(w2t_ref[...], bufh[slot], _DN,
                            preferred_element_type=jnp.float32)
            + b2_ref[...]
        )
        return carry

    jax.lax.fori_loop(0, nch, body, 0)


@jax.jit
def kernel(z_lp, z_hp, W1, b1, W2, b2):
    n, d = z_lp.shape
    w1t = W1.T
    w2t = W2.T
    b1r = b1.reshape(2, 1)
    b2r = b2.reshape(2, 1)
    nch = n // _CH
    out_shape = (
        jax.ShapeDtypeStruct((nch, 2, _CH), jnp.float32),
        jax.ShapeDtypeStruct((nch, 2, _CH), jnp.float32),
    )
    ol_t, oh_t = pl.pallas_call(
        _gates_body,
        in_specs=[
            pl.BlockSpec(memory_space=pltpu.MemorySpace.HBM),
            pl.BlockSpec(memory_space=pltpu.MemorySpace.HBM),
            pl.BlockSpec(memory_space=pltpu.MemorySpace.VMEM),
            pl.BlockSpec(memory_space=pltpu.MemorySpace.VMEM),
            pl.BlockSpec(memory_space=pltpu.MemorySpace.VMEM),
            pl.BlockSpec(memory_space=pltpu.MemorySpace.VMEM),
        ],
        out_specs=(
            pl.BlockSpec(memory_space=pltpu.MemorySpace.VMEM),
            pl.BlockSpec(memory_space=pltpu.MemorySpace.VMEM),
        ),
        out_shape=out_shape,
        scratch_shapes=[
            pltpu.VMEM((_NBUF, _CH, d), jnp.float32),
            pltpu.VMEM((_NBUF, _CH, d), jnp.float32),
            pltpu.SemaphoreType.DMA((2, _NBUF)),
        ],
    )(z_lp, z_hp, w1t, b1r, w2t, b2r)
    x_lp = ol_t.transpose(0, 2, 1).reshape(n, 2)
    x_hp = oh_t.transpose(0, 2, 1).reshape(n, 2)
    return (x_lp, x_hp)


# speedup vs baseline: 1.0076x; 1.0076x over previous
import jax
import jax.numpy as jnp
from jax import lax
from jax.experimental import pallas as pl
from jax.experimental.pallas import tpu as pltpu

_CH = 1000
_NBUF = 5

_DN = (((0,), (1,)), ((), ()))


def _gates_body(zl_hbm, zh_hbm, w1t_ref, b1_ref, w2t_ref, b2_ref,
                ol_ref, oh_ref, bufl, bufh, sems):
    n = zl_hbm.shape[0]
    nch = n // _CH

    def start(i, slot):
        pltpu.make_async_copy(
            zl_hbm.at[pl.ds(i * _CH, _CH), :], bufl.at[slot], sems.at[0, slot]
        ).start()
        pltpu.make_async_copy(
            zh_hbm.at[pl.ds(i * _CH, _CH), :], bufh.at[slot], sems.at[1, slot]
        ).start()

    for s in range(_NBUF - 1):
        start(s, s)

    def body(i, carry):
        slot = jax.lax.rem(i, _NBUF)
        nxt = i + (_NBUF - 1)

        @pl.when(nxt < nch)
        def _():
            start(nxt, jax.lax.rem(nxt, _NBUF))

        pltpu.make_async_copy(
            zl_hbm.at[pl.ds(i * _CH, _CH), :], bufl.at[slot], sems.at[0, slot]
        ).wait()
        pltpu.make_async_copy(
            zh_hbm.at[pl.ds(i * _CH, _CH), :], bufh.at[slot], sems.at[1, slot]
        ).wait()

        ol_ref[i] = (
            lax.dot_general(w1t_ref[...], bufl[slot], _DN,
                            preferred_element_type=jnp.float32)
            + b1_ref[...]
        )
        oh_ref[i] = (
            lax.dot_general(w2t_ref[...], bufh[slot], _DN,
                            preferred_element_type=jnp.float32)
            + b2_ref[...]
        )
        return carry

    jax.lax.fori_loop(0, nch, body, 0)


@jax.jit
def kernel(z_lp, z_hp, W1, b1, W2, b2):
    n, d = z_lp.shape
    w1t = W1.T
    w2t = W2.T
    b1r = b1.reshape(2, 1)
    b2r = b2.reshape(2, 1)
    nch = n // _CH
    out_shape = (
        jax.ShapeDtypeStruct((nch, 2, _CH), jnp.float32),
        jax.ShapeDtypeStruct((nch, 2, _CH), jnp.float32),
    )
    ol_t, oh_t = pl.pallas_call(
        _gates_body,
        in_specs=[
            pl.BlockSpec(memory_space=pltpu.MemorySpace.HBM),
            pl.BlockSpec(memory_space=pltpu.MemorySpace.HBM),
            pl.BlockSpec(memory_space=pltpu.MemorySpace.VMEM),
            pl.BlockSpec(memory_space=pltpu.MemorySpace.VMEM),
            pl.BlockSpec(memory_space=pltpu.MemorySpace.VMEM),
            pl.BlockSpec(memory_space=pltpu.MemorySpace.VMEM),
        ],
        out_specs=(
            pl.BlockSpec(memory_space=pltpu.MemorySpace.VMEM),
            pl.BlockSpec(memory_space=pltpu.MemorySpace.VMEM),
        ),
        out_shape=out_shape,
        scratch_shapes=[
            pltpu.VMEM((_NBUF, _CH, d), jnp.float32),
            pltpu.VMEM((_NBUF, _CH, d), jnp.float32),
            pltpu.SemaphoreType.DMA((2, _NBUF)),
        ],
    )(z_lp, z_hp, w1t, b1r, w2t, b2r)
    x_lp = ol_t.transpose(0, 2, 1).reshape(n, 2)
    x_hp = oh_t.transpose(0, 2, 1).reshape(n, 2)
    return (x_lp, x_hp)
